# baseline (device time: 175697 ns/iter reference)
import functools

import jax
import jax.numpy as jnp
from jax import lax
from jax.experimental import pallas as pl
from jax.experimental.pallas import tpu as pltpu

N_DEV = 8
M = 1024
S = 4
K = 2 * S
Q = M // K
D = 1024
N_HOP = N_DEV - 1

_signal = getattr(pl, "semaphore_signal", None) or pltpu.semaphore_signal
_swait = getattr(pl, "semaphore_wait", None) or pltpu.semaphore_wait


def kernel(partial, gamma):
    x = jnp.reshape(partial, (N_DEV * M, D))
    g2 = jnp.reshape(gamma, (1, D))

    def body(x_ref, g_ref, out_ref, cw_ref, ccw_ref, stage_ref,
             init_sems, stage_sems, send_cw, recv_cw, send_ccw, recv_ccw):
        d = lax.axis_index("i")
        left = (d + N_DEV - 1) % N_DEV
        right = (d + 1) % N_DEV

        def comm(k):
            return cw_ref if k < S else ccw_ref

        def sems(k):
            return (send_cw, recv_cw) if k < S else (send_ccw, recv_ccw)

        def chunk_idx(k, h):
            if k < S:
                return (d + 2 * N_DEV - h - 2) % N_DEV
            return (d + h + 2) % N_DEV

        def row0(k):
            return k * Q

        def x_rows(k, c):
            return x_ref.at[pl.ds(c * M + row0(k), Q), :]

        def stream(k):
            return k % S

        def rdma(k, h):
            s = stream(k)
            send_s, recv_s = sems(k)
            src = comm(k).at[s, N_DEV - 1] if h == 0 else comm(k).at[s, h - 1]
            return pltpu.make_async_remote_copy(
                src_ref=src,
                dst_ref=comm(k).at[s, h],
                send_sem=send_s.at[s, h],
                recv_sem=recv_s.at[s, h],
                device_id=((right,) if k < S else (left,)),
                device_id_type=pl.DeviceIdType.MESH,
            )

        def stage_copy(k, h):
            p = h % 2
            return pltpu.make_async_copy(
                x_rows(k, chunk_idx(k, h)),
                stage_ref.at[p * K + k],
                stage_sems.at[p * K + k],
            )

        inits = []
        for k in range(K):
            ic = pltpu.make_async_copy(
                x_rows(k, chunk_idx(k, -1)),
                comm(k).at[stream(k), N_DEV - 1],
                init_sems.at[k],
            )
            ic.start()
            inits.append(ic)

        barrier_sem = pltpu.get_barrier_semaphore()
        for nbr in (left, right):
            _signal(barrier_sem, inc=1, device_id=(nbr,),
                    device_id_type=pl.DeviceIdType.MESH)
        _swait(barrier_sem, 2)
        for k in range(K):
            inits[k].wait()
            rdma(k, 0).start()
            stage_copy(k, 0).start()

        g = g_ref[...]
        for h in range(N_HOP):
            for k in range(K):
                s = stream(k)
                rdma(k, h).wait_recv()
                stage_copy(k, h).wait()
                if h < N_HOP - 1:
                    comm(k)[s, h] = comm(k)[s, h] + stage_ref[(h % 2) * K + k]
                    rdma(k, h + 1).start()
                else:
                    y = comm(k)[s, h] + stage_ref[(h % 2) * K + k]
                    ms = jnp.sum(y * y, axis=-1, keepdims=True) * (1.0 / D)
                    out_ref[pl.ds(row0(k), Q), :] = (
                        y * lax.rsqrt(ms + 1e-6) * g
                    )
            if h < N_HOP - 1:
                for k in range(K):
                    stage_copy(k, h + 1).start()

        for h in range(N_HOP):
            for k in range(K):
                rdma(k, h).wait_send()

        @functools.partial(pl.run_scoped, sem2=pltpu.SemaphoreType.REGULAR)
        def _(sem2):
            for nbr in (left, right):
                _signal(sem2, inc=1, device_id=(nbr,),
                        device_id_type=pl.DeviceIdType.MESH)
            _swait(sem2, 2)

    return pl.pallas_call(
        body,
        out_shape=jax.ShapeDtypeStruct((M, D), jnp.float32),
        in_specs=[
            pl.BlockSpec(memory_space=pl.ANY),
            pl.BlockSpec(memory_space=pltpu.VMEM),
        ],
        out_specs=pl.BlockSpec(memory_space=pltpu.VMEM),
        scratch_shapes=[
            pltpu.VMEM((S, N_DEV, Q, D), jnp.float32),
            pltpu.VMEM((S, N_DEV, Q, D), jnp.float32),
            pltpu.VMEM((2 * K, Q, D), jnp.float32),
            pltpu.SemaphoreType.DMA((K,)),
            pltpu.SemaphoreType.DMA((2 * K,)),
            pltpu.SemaphoreType.DMA((S, N_HOP)),
            pltpu.SemaphoreType.DMA((S, N_HOP)),
            pltpu.SemaphoreType.DMA((S, N_HOP)),
            pltpu.SemaphoreType.DMA((S, N_HOP)),
        ],
        compiler_params=pltpu.CompilerParams(
            collective_id=0, vmem_limit_bytes=60 * 1024 * 1024,
        ),
    )(x, g2)


# device time: 175617 ns/iter; 1.0005x vs baseline; 1.0005x over previous
import functools

import jax
import jax.numpy as jnp
from jax import lax
from jax.experimental import pallas as pl
from jax.experimental.pallas import tpu as pltpu

N_DEV = 8
M = 1024
S = 4
K = 2 * S
Q = M // K
D = 1024
N_HOP = N_DEV - 1

_signal = getattr(pl, "semaphore_signal", None) or pltpu.semaphore_signal
_swait = getattr(pl, "semaphore_wait", None) or pltpu.semaphore_wait


def kernel(partial, gamma):
    x = partial
    g2 = jnp.reshape(gamma, (1, D))

    def body(x_ref, g_ref, out_ref, cw_ref, ccw_ref, stage_ref,
             init_sems, stage_sems, send_cw, recv_cw, send_ccw, recv_ccw):
        d = lax.axis_index("i")
        left = (d + N_DEV - 1) % N_DEV
        right = (d + 1) % N_DEV

        def comm(k):
            return cw_ref if k < S else ccw_ref

        def sems(k):
            return (send_cw, recv_cw) if k < S else (send_ccw, recv_ccw)

        def chunk_idx(k, h):
            if k < S:
                return (d + 2 * N_DEV - h - 2) % N_DEV
            return (d + h + 2) % N_DEV

        def row0(k):
            return k * Q

        def x_rows(k, c):
            return x_ref.at[0, pl.ds(c * M + row0(k), Q), :]

        def stream(k):
            return k % S

        def rdma(k, h):
            s = stream(k)
            send_s, recv_s = sems(k)
            src = comm(k).at[s, N_DEV - 1] if h == 0 else comm(k).at[s, h - 1]
            return pltpu.make_async_remote_copy(
                src_ref=src,
                dst_ref=comm(k).at[s, h],
                send_sem=send_s.at[s, h],
                recv_sem=recv_s.at[s, h],
                device_id=((right,) if k < S else (left,)),
                device_id_type=pl.DeviceIdType.MESH,
            )

        def stage_copy(k, h):
            p = h % 2
            return pltpu.make_async_copy(
                x_rows(k, chunk_idx(k, h)),
                stage_ref.at[p * K + k],
                stage_sems.at[p * K + k],
            )

        inits = []
        for k in range(K):
            ic = pltpu.make_async_copy(
                x_rows(k, chunk_idx(k, -1)),
                comm(k).at[stream(k), N_DEV - 1],
                init_sems.at[k],
            )
            ic.start()
            inits.append(ic)

        barrier_sem = pltpu.get_barrier_semaphore()
        for nbr in (left, right):
            _signal(barrier_sem, inc=1, device_id=(nbr,),
                    device_id_type=pl.DeviceIdType.MESH)
        _swait(barrier_sem, 2)
        for k in range(K):
            inits[k].wait()
            rdma(k, 0).start()
            stage_copy(k, 0).start()

        g = g_ref[...]
        for h in range(N_HOP):
            for k in range(K):
                s = stream(k)
                rdma(k, h).wait_recv()
                stage_copy(k, h).wait()
                if h < N_HOP - 1:
                    comm(k)[s, h] = comm(k)[s, h] + stage_ref[(h % 2) * K + k]
                    rdma(k, h + 1).start()
                else:
                    y = comm(k)[s, h] + stage_ref[(h % 2) * K + k]
                    ms = jnp.sum(y * y, axis=-1, keepdims=True) * (1.0 / D)
                    out_ref[pl.ds(row0(k), Q), :] = (
                        y * lax.rsqrt(ms + 1e-6) * g
                    )
            if h < N_HOP - 1:
                for k in range(K):
                    stage_copy(k, h + 1).start()

        for h in range(N_HOP):
            for k in range(K):
                rdma(k, h).wait_send()

        @functools.partial(pl.run_scoped, sem2=pltpu.SemaphoreType.REGULAR)
        def _(sem2):
            for nbr in (left, right):
                _signal(sem2, inc=1, device_id=(nbr,),
                        device_id_type=pl.DeviceIdType.MESH)
            _swait(sem2, 2)

    return pl.pallas_call(
        body,
        out_shape=jax.ShapeDtypeStruct((M, D), jnp.float32),
        in_specs=[
            pl.BlockSpec(memory_space=pl.ANY),
            pl.BlockSpec(memory_space=pltpu.VMEM),
        ],
        out_specs=pl.BlockSpec(memory_space=pltpu.VMEM),
        scratch_shapes=[
            pltpu.VMEM((S, N_DEV, Q, D), jnp.float32),
            pltpu.VMEM((S, N_DEV, Q, D), jnp.float32),
            pltpu.VMEM((2 * K, Q, D), jnp.float32),
            pltpu.SemaphoreType.DMA((K,)),
            pltpu.SemaphoreType.DMA((2 * K,)),
            pltpu.SemaphoreType.DMA((S, N_HOP)),
            pltpu.SemaphoreType.DMA((S, N_HOP)),
            pltpu.SemaphoreType.DMA((S, N_HOP)),
            pltpu.SemaphoreType.DMA((S, N_HOP)),
        ],
        compiler_params=pltpu.CompilerParams(
            collective_id=0, vmem_limit_bytes=60 * 1024 * 1024,
        ),
    )(x, g2)


# device time: 174683 ns/iter; 1.0058x vs baseline; 1.0053x over previous
import functools

import jax
import jax.numpy as jnp
from jax import lax
from jax.experimental import pallas as pl
from jax.experimental.pallas import tpu as pltpu

N_DEV = 8
M = 1024
S = 4
K = 2 * S
Q = M // K
D = 1024
N_HOP = N_DEV - 1

_signal = getattr(pl, "semaphore_signal", None) or pltpu.semaphore_signal
_swait = getattr(pl, "semaphore_wait", None) or pltpu.semaphore_wait


def kernel(partial, gamma):
    x = partial
    g2 = jnp.reshape(gamma, (1, D))

    def body(x_ref, g_ref, out_ref, cw_ref, ccw_ref, stage_ref, ostage_ref,
             init_sems, stage_sems, out_sems,
             send_cw, recv_cw, send_ccw, recv_ccw):
        d = lax.axis_index("i")
        left = (d + N_DEV - 1) % N_DEV
        right = (d + 1) % N_DEV

        def comm(k):
            return cw_ref if k < S else ccw_ref

        def sems(k):
            return (send_cw, recv_cw) if k < S else (send_ccw, recv_ccw)

        def chunk_idx(k, h):
            if k < S:
                return (d + 2 * N_DEV - h - 2) % N_DEV
            return (d + h + 2) % N_DEV

        def row0(k):
            return k * Q

        def x_rows(k, c):
            return x_ref.at[0, pl.ds(c * M + row0(k), Q), :]

        def stream(k):
            return k % S

        def rdma(k, h):
            s = stream(k)
            send_s, recv_s = sems(k)
            src = comm(k).at[s, N_DEV - 1] if h == 0 else comm(k).at[s, h - 1]
            return pltpu.make_async_remote_copy(
                src_ref=src,
                dst_ref=comm(k).at[s, h],
                send_sem=send_s.at[s, h],
                recv_sem=recv_s.at[s, h],
                device_id=((right,) if k < S else (left,)),
                device_id_type=pl.DeviceIdType.MESH,
            )

        def stage_copy(k, h):
            p = h % 2
            return pltpu.make_async_copy(
                x_rows(k, chunk_idx(k, h)),
                stage_ref.at[p * K + k],
                stage_sems.at[p * K + k],
            )

        inits = []
        for k in range(K):
            ic = pltpu.make_async_copy(
                x_rows(k, chunk_idx(k, -1)),
                comm(k).at[stream(k), N_DEV - 1],
                init_sems.at[k],
            )
            ic.start()
            inits.append(ic)

        barrier_sem = pltpu.get_barrier_semaphore()
        for nbr in (left, right):
            _signal(barrier_sem, inc=1, device_id=(nbr,),
                    device_id_type=pl.DeviceIdType.MESH)
        _swait(barrier_sem, 2)
        for k in range(K):
            inits[k].wait()
            rdma(k, 0).start()
            stage_copy(k, 0).start()

        g = g_ref[...]
        for h in range(N_HOP):
            for k in range(K):
                s = stream(k)
                rdma(k, h).wait_recv()
                stage_copy(k, h).wait()
                if h < N_HOP - 1:
                    comm(k)[s, h] = comm(k)[s, h] + stage_ref[(h % 2) * K + k]
                    rdma(k, h + 1).start()
                else:
                    y = comm(k)[s, h] + stage_ref[(h % 2) * K + k]
                    ms = jnp.sum(y * y, axis=-1, keepdims=True) * (1.0 / D)
                    ostage_ref[k] = y * lax.rsqrt(ms + 1e-6) * g
                    pltpu.make_async_copy(
                        ostage_ref.at[k],
                        out_ref.at[pl.ds(row0(k), Q), :],
                        out_sems.at[k],
                    ).start()
            if h < N_HOP - 1:
                for k in range(K):
                    stage_copy(k, h + 1).start()

        for k in range(K):
            pltpu.make_async_copy(
                ostage_ref.at[k],
                out_ref.at[pl.ds(row0(k), Q), :],
                out_sems.at[k],
            ).wait()
        for h in range(N_HOP):
            for k in range(K):
                rdma(k, h).wait_send()

        @functools.partial(pl.run_scoped, sem2=pltpu.SemaphoreType.REGULAR)
        def _(sem2):
            for nbr in (left, right):
                _signal(sem2, inc=1, device_id=(nbr,),
                        device_id_type=pl.DeviceIdType.MESH)
            _swait(sem2, 2)

    return pl.pallas_call(
        body,
        out_shape=jax.ShapeDtypeStruct((M, D), jnp.float32),
        in_specs=[
            pl.BlockSpec(memory_space=pl.ANY),
            pl.BlockSpec(memory_space=pltpu.VMEM),
        ],
        out_specs=pl.BlockSpec(memory_space=pl.ANY),
        scratch_shapes=[
            pltpu.VMEM((S, N_DEV, Q, D), jnp.float32),
            pltpu.VMEM((S, N_DEV, Q, D), jnp.float32),
            pltpu.VMEM((2 * K, Q, D), jnp.float32),
            pltpu.VMEM((K, Q, D), jnp.float32),
            pltpu.SemaphoreType.DMA((K,)),
            pltpu.SemaphoreType.DMA((2 * K,)),
            pltpu.SemaphoreType.DMA((K,)),
            pltpu.SemaphoreType.DMA((S, N_HOP)),
            pltpu.SemaphoreType.DMA((S, N_HOP)),
            pltpu.SemaphoreType.DMA((S, N_HOP)),
            pltpu.SemaphoreType.DMA((S, N_HOP)),
        ],
        compiler_params=pltpu.CompilerParams(
            collective_id=0, vmem_limit_bytes=60 * 1024 * 1024,
        ),
    )(x, g2)
